# baseline (speedup 1.0000x reference)
import functools

import jax
import jax.numpy as jnp
from jax import lax
from jax.experimental import pallas as pl
from jax.experimental.pallas import tpu as pltpu
from jax.experimental.pallas import tpu_sc as plsc

_C0 = 20000
_C1 = 60000
_C2 = 100000
_D0, _D1, _D2 = 1024, 256, 64
_D2P = 256
_OUT = 1024
_W0, _W1, _W2 = _D0 // 2, _D1 // 2, _D2P // 2

_NC, _NS, _L = 2, 16, 16
_NW = _NC * _NS
_B = 8192
_BPW = _B // _NW
_CHUNK = 64
_NCHUNK = _BPW // _CHUNK


def _sc_gather_body(ids_hbm, emb0, emb1, emb2, x0_hbm, x1_hbm, x2_hbm,
                    ids_v, i0_v, i1_v, i2_v,
                    r0a, r0b, r1a, r1b, r2a, r2b, gsem, wsem):
    wid = lax.axis_index("s") * _NC + lax.axis_index("c")
    base = wid * _BPW
    pltpu.sync_copy(ids_hbm.at[pl.ds(base, _BPW)], ids_v)
    for c in range(_NCHUNK):
        for j in range(_CHUNK // _L):
            t = ids_v[pl.ds(c * _CHUNK + j * _L, _L)]
            i0_v[c, pl.ds(j * _L, _L)] = jnp.clip(t, 0, _C0 - 1)
            i1_v[c, pl.ds(j * _L, _L)] = jnp.clip(t - _C0, 0, (_C1 - _C0) - 1)
            i2_v[c, pl.ds(j * _L, _L)] = jnp.clip(t - _C1, 0, (_C2 - _C1) - 1)
    r0 = (r0a, r0b)
    r1 = (r1a, r1b)
    r2 = (r2a, r2b)

    def fire_gather(c, b):
        return (pltpu.async_copy(emb0.at[i0_v.at[c]], r0[b], gsem),
                pltpu.async_copy(emb1.at[i1_v.at[c]], r1[b], gsem),
                pltpu.async_copy(emb2.at[i2_v.at[c]], r2[b], gsem))

    def fire_write(c, b):
        off = base + c * _CHUNK
        return (pltpu.async_copy(r0[b], x0_hbm.at[pl.ds(off, _CHUNK)], wsem),
                pltpu.async_copy(r1[b], x1_hbm.at[pl.ds(off, _CHUNK)], wsem),
                pltpu.async_copy(r2[b], x2_hbm.at[pl.ds(off, _CHUNK)], wsem))

    g = [None] * _NCHUNK
    w = [None] * _NCHUNK
    g[0] = fire_gather(0, 0)
    for c in range(_NCHUNK):
        if c + 1 < _NCHUNK:
            if c >= 1:
                for cp in w[c - 1]:
                    cp.wait()
            g[c + 1] = fire_gather(c + 1, (c + 1) % 2)
        for cp in g[c]:
            cp.wait()
        w[c] = fire_write(c, c % 2)
    for cp in w[_NCHUNK - 2]:
        cp.wait()
    for cp in w[_NCHUNK - 1]:
        cp.wait()


_sc_gather = pl.kernel(
    _sc_gather_body,
    out_type=(
        jax.ShapeDtypeStruct((_B, _W0), jnp.int32),
        jax.ShapeDtypeStruct((_B, _W1), jnp.int32),
        jax.ShapeDtypeStruct((_B, _W2), jnp.int32),
    ),
    mesh=plsc.VectorSubcoreMesh(core_axis_name="c", subcore_axis_name="s"),
    scratch_types=[
        pltpu.VMEM((_BPW,), jnp.int32),
        pltpu.VMEM((_NCHUNK, _CHUNK), jnp.int32),
        pltpu.VMEM((_NCHUNK, _CHUNK), jnp.int32),
        pltpu.VMEM((_NCHUNK, _CHUNK), jnp.int32),
        pltpu.VMEM((_CHUNK, _W0), jnp.int32),
        pltpu.VMEM((_CHUNK, _W0), jnp.int32),
        pltpu.VMEM((_CHUNK, _W1), jnp.int32),
        pltpu.VMEM((_CHUNK, _W1), jnp.int32),
        pltpu.VMEM((_CHUNK, _W2), jnp.int32),
        pltpu.VMEM((_CHUNK, _W2), jnp.int32),
        pltpu.SemaphoreType.DMA,
        pltpu.SemaphoreType.DMA,
    ],
)

_BT = 512


def _tc_body(ids_ref, x0_ref, x1_ref, x2_ref, w0_ref, w1_ref, w2_ref, out_ref):
    t = ids_ref[...]
    m0 = (t < _C0).astype(jnp.bfloat16)
    m1 = jnp.logical_and(t >= _C0, t < _C1).astype(jnp.bfloat16)
    m2 = (t >= _C1).astype(jnp.bfloat16)
    acc = jnp.dot(x0_ref[...] * m0, w0_ref[...], preferred_element_type=jnp.float32)
    acc += jnp.dot(x1_ref[...] * m1, w1_ref[...], preferred_element_type=jnp.float32)
    acc += jnp.dot(x2_ref[...] * m2, w2_ref[...], preferred_element_type=jnp.float32)
    out_ref[...] = acc


def _pack(a):
    b = a.astype(jnp.bfloat16).reshape(a.shape[0], -1, 2)
    return lax.bitcast_convert_type(b, jnp.int32)


def _unpack(a, d):
    return lax.bitcast_convert_type(a, jnp.bfloat16).reshape(a.shape[0], d)


@functools.partial(jax.jit, static_argnames=())
def _run(ids, emb0p, w0t, emb1p, w1t, emb2p, w2t):
    x0, x1, x2 = _sc_gather(ids, emb0p, emb1p, emb2p)
    x0 = _unpack(x0, _D0)
    x1 = _unpack(x1, _D1)
    x2 = _unpack(x2, _D2P)
    ids2d = ids.reshape(_B, 1)
    grid = _B // _BT
    out = pl.pallas_call(
        _tc_body,
        grid=(grid,),
        in_specs=[
            pl.BlockSpec((_BT, 1), lambda i: (i, 0)),
            pl.BlockSpec((_BT, _D0), lambda i: (i, 0)),
            pl.BlockSpec((_BT, _D1), lambda i: (i, 0)),
            pl.BlockSpec((_BT, _D2P), lambda i: (i, 0)),
            pl.BlockSpec((_D0, _OUT), lambda i: (0, 0)),
            pl.BlockSpec((_D1, _OUT), lambda i: (0, 0)),
            pl.BlockSpec((_D2P, _OUT), lambda i: (0, 0)),
        ],
        out_specs=pl.BlockSpec((_BT, _OUT), lambda i: (i, 0)),
        out_shape=jax.ShapeDtypeStruct((_B, _OUT), jnp.float32),
    )(ids2d, x0, x1, x2, w0t, w1t, w2t)
    return out


def kernel(input, emb0, W0, emb1, W1, emb2, W2):
    ids = input.reshape(-1).astype(jnp.int32)
    emb2pad = jnp.pad(emb2, ((0, 0), (0, _D2P - _D2)))
    w2tpad = jnp.pad(W2.T, ((0, _D2P - _D2), (0, 0)))
    out = _run(ids,
               _pack(emb0), W0.T.astype(jnp.bfloat16),
               _pack(emb1), W1.T.astype(jnp.bfloat16),
               _pack(emb2pad), w2tpad.astype(jnp.bfloat16))
    return out.reshape(input.shape + (_OUT,))


# speedup vs baseline: 3.9317x; 3.9317x over previous
import functools

import jax
import jax.numpy as jnp
from jax import lax
from jax.experimental import pallas as pl
from jax.experimental.pallas import tpu as pltpu
from jax.experimental.pallas import tpu_sc as plsc

_C0 = 20000
_C1 = 60000
_C2 = 100000
_D0, _D1, _D2 = 1024, 256, 64
_D2P = 128
_OUT = 1024

_NC, _NS, _L = 2, 16, 16
_NW = _NC * _NS
_B = 8192
_NSTAGE = 4
_BS = _B // _NSTAGE
_BPW = _BS // _NW
_CHUNK = 32
_NCHUNK = _BPW // _CHUNK


def _sc_gather_body(ids_hbm, emb0, emb1, emb2, x0_hbm, x1_hbm, x2_hbm,
                    ids_v, i0_v, i1_v, i2_v,
                    r0a, r0b, r1a, r1b, r2a, r2b, gsem, wsem):
    wid = lax.axis_index("s") * _NC + lax.axis_index("c")
    base = wid * _BPW
    pltpu.sync_copy(ids_hbm.at[pl.ds(base, _BPW)], ids_v)
    for c in range(_NCHUNK):
        for j in range(_CHUNK // _L):
            t = ids_v[pl.ds(c * _CHUNK + j * _L, _L)]
            i0_v[c, pl.ds(j * _L, _L)] = jnp.clip(t, 0, _C0 - 1)
            i1_v[c, pl.ds(j * _L, _L)] = jnp.clip(t - _C0, 0, (_C1 - _C0) - 1)
            i2_v[c, pl.ds(j * _L, _L)] = jnp.clip(t - _C1, 0, (_C2 - _C1) - 1)
    r0 = (r0a, r0b)
    r1 = (r1a, r1b)
    r2 = (r2a, r2b)

    def fire_gather(c, b):
        return (pltpu.async_copy(emb0.at[i0_v.at[c]], r0[b], gsem),
                pltpu.async_copy(emb1.at[i1_v.at[c]], r1[b], gsem),
                pltpu.async_copy(emb2.at[i2_v.at[c]], r2[b], gsem))

    def fire_write(c, b):
        off = base + c * _CHUNK
        return (pltpu.async_copy(r0[b], x0_hbm.at[pl.ds(off, _CHUNK)], wsem),
                pltpu.async_copy(r1[b], x1_hbm.at[pl.ds(off, _CHUNK)], wsem),
                pltpu.async_copy(r2[b], x2_hbm.at[pl.ds(off, _CHUNK)], wsem))

    g = [None] * _NCHUNK
    w = [None] * _NCHUNK
    g[0] = fire_gather(0, 0)
    for c in range(_NCHUNK):
        if c + 1 < _NCHUNK:
            if c >= 1:
                for cp in w[c - 1]:
                    cp.wait()
            g[c + 1] = fire_gather(c + 1, (c + 1) % 2)
        for cp in g[c]:
            cp.wait()
        w[c] = fire_write(c, c % 2)
    for cp in w[_NCHUNK - 2]:
        cp.wait()
    for cp in w[_NCHUNK - 1]:
        cp.wait()


_sc_gather = pl.kernel(
    _sc_gather_body,
    out_type=(
        jax.ShapeDtypeStruct((_BS, _D0), jnp.float32),
        jax.ShapeDtypeStruct((_BS, _D1), jnp.float32),
        jax.ShapeDtypeStruct((_BS, _D2P), jnp.float32),
    ),
    mesh=plsc.VectorSubcoreMesh(core_axis_name="c", subcore_axis_name="s"),
    scratch_types=[
        pltpu.VMEM((_BPW,), jnp.int32),
        pltpu.VMEM((_NCHUNK, _CHUNK), jnp.int32),
        pltpu.VMEM((_NCHUNK, _CHUNK), jnp.int32),
        pltpu.VMEM((_NCHUNK, _CHUNK), jnp.int32),
        pltpu.VMEM((_CHUNK, _D0), jnp.float32),
        pltpu.VMEM((_CHUNK, _D0), jnp.float32),
        pltpu.VMEM((_CHUNK, _D1), jnp.float32),
        pltpu.VMEM((_CHUNK, _D1), jnp.float32),
        pltpu.VMEM((_CHUNK, _D2P), jnp.float32),
        pltpu.VMEM((_CHUNK, _D2P), jnp.float32),
        pltpu.SemaphoreType.DMA,
        pltpu.SemaphoreType.DMA,
    ],
)

_BT = 512


def _tc_body(ids_ref, x0_ref, x1_ref, x2_ref, w0_ref, w1_ref, w2_ref, out_ref):
    t = ids_ref[...]
    m0 = (t < _C0).astype(jnp.float32)
    m1 = jnp.logical_and(t >= _C0, t < _C1).astype(jnp.float32)
    m2 = (t >= _C1).astype(jnp.float32)
    acc = jnp.dot(x0_ref[...] * m0, w0_ref[...], preferred_element_type=jnp.float32)
    acc += jnp.dot(x1_ref[...] * m1, w1_ref[...], preferred_element_type=jnp.float32)
    acc += jnp.dot(x2_ref[...] * m2, w2_ref[...], preferred_element_type=jnp.float32)
    out_ref[...] = acc


def _tc_call(ids_s, x0, x1, x2, w0t, w1t, w2t):
    ids2d = ids_s.reshape(_BS, 1)
    grid = _BS // _BT
    return pl.pallas_call(
        _tc_body,
        grid=(grid,),
        in_specs=[
            pl.BlockSpec((_BT, 1), lambda i: (i, 0)),
            pl.BlockSpec((_BT, _D0), lambda i: (i, 0)),
            pl.BlockSpec((_BT, _D1), lambda i: (i, 0)),
            pl.BlockSpec((_BT, _D2P), lambda i: (i, 0)),
            pl.BlockSpec((_D0, _OUT), lambda i: (0, 0)),
            pl.BlockSpec((_D1, _OUT), lambda i: (0, 0)),
            pl.BlockSpec((_D2P, _OUT), lambda i: (0, 0)),
        ],
        out_specs=pl.BlockSpec((_BT, _OUT), lambda i: (i, 0)),
        out_shape=jax.ShapeDtypeStruct((_BS, _OUT), jnp.float32),
    )(ids2d, x0, x1, x2, w0t, w1t, w2t)


@functools.partial(jax.jit, static_argnames=())
def _run(ids, emb0, w0t, emb1, w1t, emb2, w2t):
    outs = []
    for s in range(_NSTAGE):
        ids_s = lax.dynamic_slice_in_dim(ids, s * _BS, _BS)
        x0, x1, x2 = _sc_gather(ids_s, emb0, emb1, emb2)
        outs.append(_tc_call(ids_s, x0, x1, x2, w0t, w1t, w2t))
    return jnp.concatenate(outs, axis=0)


def kernel(input, emb0, W0, emb1, W1, emb2, W2):
    ids = input.reshape(-1).astype(jnp.int32)
    emb2p = jnp.pad(emb2, ((0, 0), (0, _D2P - _D2)))
    w2tp = jnp.pad(W2.T, ((0, _D2P - _D2), (0, 0)))
    out = _run(ids, emb0, W0.T, emb1, W1.T, emb2p, w2tp)
    return out.reshape(input.shape + (_OUT,))


# speedup vs baseline: 4.4368x; 1.1285x over previous
import functools

import jax
import jax.numpy as jnp
from jax import lax
from jax.experimental import pallas as pl
from jax.experimental.pallas import tpu as pltpu
from jax.experimental.pallas import tpu_sc as plsc

_C0 = 20000
_C1 = 60000
_C2 = 100000
_D0, _D1, _D2 = 1024, 256, 64
_D2P = 128
_OUT = 1024

_NC, _NS, _L = 2, 16, 16
_NW = _NC * _NS
_B = 8192
_BPW = _B // _NW
_CHUNK = 32
_NCHUNK = _BPW // _CHUNK


def _sc_gather_body(ids_hbm, emb0, emb1, emb2, x0_hbm, x1_hbm, x2_hbm,
                    ids_v, i0_v, i1_v, i2_v,
                    r0a, r0b, r1a, r1b, r2a, r2b, gsem, wsem):
    wid = lax.axis_index("s") * _NC + lax.axis_index("c")
    base = wid * _BPW
    pltpu.sync_copy(ids_hbm.at[pl.ds(base, _BPW)], ids_v)
    for c in range(_NCHUNK):
        for j in range(_CHUNK // _L):
            t = ids_v[pl.ds(c * _CHUNK + j * _L, _L)]
            i0_v[c, pl.ds(j * _L, _L)] = jnp.clip(t, 0, _C0 - 1)
            i1_v[c, pl.ds(j * _L, _L)] = jnp.clip(t - _C0, 0, (_C1 - _C0) - 1)
            i2_v[c, pl.ds(j * _L, _L)] = jnp.clip(t - _C1, 0, (_C2 - _C1) - 1)
    r0 = (r0a, r0b)
    r1 = (r1a, r1b)
    r2 = (r2a, r2b)

    def fire_gather(c, b):
        return (pltpu.async_copy(emb0.at[i0_v.at[c]], r0[b], gsem),
                pltpu.async_copy(emb1.at[i1_v.at[c]], r1[b], gsem),
                pltpu.async_copy(emb2.at[i2_v.at[c]], r2[b], gsem))

    def fire_write(c, b):
        off = base + c * _CHUNK
        return (pltpu.async_copy(r0[b], x0_hbm.at[pl.ds(off, _CHUNK)], wsem),
                pltpu.async_copy(r1[b], x1_hbm.at[pl.ds(off, _CHUNK)], wsem),
                pltpu.async_copy(r2[b], x2_hbm.at[pl.ds(off, _CHUNK)], wsem))

    g = [None] * _NCHUNK
    w = [None] * _NCHUNK
    g[0] = fire_gather(0, 0)
    for c in range(_NCHUNK):
        if c + 1 < _NCHUNK:
            if c >= 1:
                for cp in w[c - 1]:
                    cp.wait()
            g[c + 1] = fire_gather(c + 1, (c + 1) % 2)
        for cp in g[c]:
            cp.wait()
        w[c] = fire_write(c, c % 2)
    for cp in w[_NCHUNK - 2]:
        cp.wait()
    for cp in w[_NCHUNK - 1]:
        cp.wait()


_sc_gather = pl.kernel(
    _sc_gather_body,
    out_type=(
        jax.ShapeDtypeStruct((_B, _D0), jnp.float32),
        jax.ShapeDtypeStruct((_B, _D1), jnp.float32),
        jax.ShapeDtypeStruct((_B, _D2P), jnp.float32),
    ),
    mesh=plsc.VectorSubcoreMesh(core_axis_name="c", subcore_axis_name="s"),
    scratch_types=[
        pltpu.VMEM((_BPW,), jnp.int32),
        pltpu.VMEM((_NCHUNK, _CHUNK), jnp.int32),
        pltpu.VMEM((_NCHUNK, _CHUNK), jnp.int32),
        pltpu.VMEM((_NCHUNK, _CHUNK), jnp.int32),
        pltpu.VMEM((_CHUNK, _D0), jnp.float32),
        pltpu.VMEM((_CHUNK, _D0), jnp.float32),
        pltpu.VMEM((_CHUNK, _D1), jnp.float32),
        pltpu.VMEM((_CHUNK, _D1), jnp.float32),
        pltpu.VMEM((_CHUNK, _D2P), jnp.float32),
        pltpu.VMEM((_CHUNK, _D2P), jnp.float32),
        pltpu.SemaphoreType.DMA,
        pltpu.SemaphoreType.DMA,
    ],
)

_BT = 512


def _tc_body(ids_ref, x0_ref, x1_ref, x2_ref, w0_ref, w1_ref, w2_ref, out_ref):
    t = ids_ref[...]
    m0 = (t < _C0).astype(jnp.bfloat16)
    m1 = jnp.logical_and(t >= _C0, t < _C1).astype(jnp.bfloat16)
    m2 = (t >= _C1).astype(jnp.bfloat16)
    a0 = x0_ref[...].astype(jnp.bfloat16) * m0
    a1 = x1_ref[...].astype(jnp.bfloat16) * m1
    a2 = x2_ref[...].astype(jnp.bfloat16) * m2
    acc = jnp.dot(a0, w0_ref[...], preferred_element_type=jnp.float32)
    acc += jnp.dot(a1, w1_ref[...], preferred_element_type=jnp.float32)
    acc += jnp.dot(a2, w2_ref[...], preferred_element_type=jnp.float32)
    out_ref[...] = acc


@functools.partial(jax.jit, static_argnames=())
def _run(ids, emb0, w0t, emb1, w1t, emb2, w2t):
    x0, x1, x2 = _sc_gather(ids, emb0, emb1, emb2)
    ids2d = ids.reshape(_B, 1)
    grid = _B // _BT
    out = pl.pallas_call(
        _tc_body,
        grid=(grid,),
        in_specs=[
            pl.BlockSpec((_BT, 1), lambda i: (i, 0)),
            pl.BlockSpec((_BT, _D0), lambda i: (i, 0)),
            pl.BlockSpec((_BT, _D1), lambda i: (i, 0)),
            pl.BlockSpec((_BT, _D2P), lambda i: (i, 0)),
            pl.BlockSpec((_D0, _OUT), lambda i: (0, 0)),
            pl.BlockSpec((_D1, _OUT), lambda i: (0, 0)),
            pl.BlockSpec((_D2P, _OUT), lambda i: (0, 0)),
        ],
        out_specs=pl.BlockSpec((_BT, _OUT), lambda i: (i, 0)),
        out_shape=jax.ShapeDtypeStruct((_B, _OUT), jnp.float32),
    )(ids2d, x0, x1, x2, w0t, w1t, w2t)
    return out


def kernel(input, emb0, W0, emb1, W1, emb2, W2):
    ids = input.reshape(-1).astype(jnp.int32)
    emb2p = jnp.pad(emb2, ((0, 0), (0, _D2P - _D2)))
    w2tp = jnp.pad(W2.T, ((0, _D2P - _D2), (0, 0)))
    out = _run(ids, emb0, W0.T.astype(jnp.bfloat16), emb1,
               W1.T.astype(jnp.bfloat16), emb2p, w2tp.astype(jnp.bfloat16))
    return out.reshape(input.shape + (_OUT,))
